# baseline (speedup 1.0000x reference)
import functools

import jax
import jax.numpy as jnp
from jax import lax
from jax.experimental import pallas as pl
from jax.experimental.pallas import tpu as pltpu
from jax.experimental.pallas import tpu_sc as plsc

NC = 2
NS = 16
NW = NC * NS
CHUNK = 128


def _gather_rows(table, ids, emb):
    n = ids.shape[0]
    per_w = n // NW
    ch = per_w // CHUNK
    ids3 = ids.reshape(NW, ch, CHUNK)
    mesh = plsc.VectorSubcoreMesh(core_axis_name="c", subcore_axis_name="s")

    @functools.partial(
        pl.kernel,
        mesh=mesh,
        out_type=jax.ShapeDtypeStruct((n, emb), jnp.float32),
        scratch_types=[
            pltpu.VMEM((ch, CHUNK), jnp.int32),
            pltpu.VMEM((CHUNK, emb), jnp.float32),
            pltpu.VMEM((CHUNK, emb), jnp.float32),
            pltpu.SemaphoreType.DMA,
            pltpu.SemaphoreType.DMA,
        ],
    )
    def k(table_hbm, idx_hbm, out_hbm, idx_v, buf0, buf1, sem0, sem1):
        wid = lax.axis_index("s") * NC + lax.axis_index("c")
        pltpu.sync_copy(idx_hbm.at[wid], idx_v)
        base = wid * per_w
        pltpu.async_copy(table_hbm.at[idx_v.at[0]], buf0, sem0)

        def pair(i, carry):
            j = 2 * i
            pltpu.async_copy(table_hbm.at[idx_v.at[j + 1]], buf1, sem1)
            pltpu.make_async_copy(table_hbm.at[idx_v.at[j]], buf0, sem0).wait()
            pltpu.sync_copy(buf0, out_hbm.at[pl.ds(base + j * CHUNK, CHUNK)])

            @pl.when(j + 2 < ch)
            def _():
                pltpu.async_copy(table_hbm.at[idx_v.at[j + 2]], buf0, sem0)

            pltpu.make_async_copy(table_hbm.at[idx_v.at[j + 1]], buf1, sem1).wait()
            pltpu.sync_copy(buf1, out_hbm.at[pl.ds(base + (j + 1) * CHUNK, CHUNK)])
            return carry

        lax.fori_loop(0, ch // 2, pair, 0)

    return k(table, ids3)


def _project(we, tt3, p2, dtg, wg, prev, c0, B):
    Bc, S, emb = we.shape
    hid = wg.shape[1]

    BB = 8

    def body(we_ref, tt_ref, p2_ref, dtg_ref, w_ref, *rest):
        out_ref = rest[-1]
        zero = jnp.zeros((1, emb), jnp.bfloat16)
        for q in range(BB):
            x = we_ref[q].astype(jnp.bfloat16)
            up = jnp.concatenate([x[1:], zero], axis=0)
            down = jnp.concatenate([zero, x[:-1]], axis=0)
            cat = jnp.concatenate([up, x, down], axis=1)
            acc = jnp.dot(cat, w_ref[...], preferred_element_type=jnp.float32)
            ttf = tt_ref[q].astype(jnp.float32)
            out_ref[q] = acc + ttf * dtg_ref[...] + p2_ref[...]

    in_specs = [
        pl.BlockSpec((BB, S, emb), lambda i: (i, 0, 0)),
        pl.BlockSpec((BB, S, 1), lambda i: (i, 0, 0)),
        pl.BlockSpec((S, hid), lambda i: (0, 0)),
        pl.BlockSpec((1, hid), lambda i: (0, 0)),
        pl.BlockSpec(wg.shape, lambda i: (0, 0)),
    ]
    inputs = [we, tt3, p2, dtg, wg]
    aliases = {}
    if prev is not None:
        in_specs.append(pl.BlockSpec(memory_space=pl.ANY))
        inputs.append(prev)
        aliases = {5: 0}
    return pl.pallas_call(
        body,
        grid=(Bc // BB,),
        in_specs=in_specs,
        out_specs=pl.BlockSpec((BB, S, hid), lambda i: (c0 + i, 0, 0)),
        out_shape=jax.ShapeDtypeStruct((B, S, hid), jnp.float32),
        input_output_aliases=aliases,
    )(*inputs)


def kernel(input_ids, token_type_ids, word_table, pos_table, type_table, W, b,
           ln_weight, ln_bias):
    B, S = input_ids.shape
    emb = word_table.shape[1]
    hid = W.shape[1]
    ids = input_ids.astype(jnp.int32)
    tt3 = token_type_ids.astype(jnp.int32).reshape(B, S, 1)
    g = ln_weight.reshape(1, hid)
    p2 = (pos_table + b.reshape(1, hid) + type_table[0].reshape(1, hid)) * g \
        + ln_bias.reshape(1, hid)
    dtg = ((type_table[1] - type_table[0]).reshape(1, hid) * g)
    wg = (W * ln_weight.reshape(1, hid)).astype(jnp.bfloat16)

    NCHUNK = 4
    BB = 8
    Bc = B // NCHUNK
    wes = [
        _gather_rows(word_table, ids[i * Bc:(i + 1) * Bc].reshape(Bc * S), emb)
        .reshape(Bc, S, emb)
        for i in range(NCHUNK)
    ]
    out = None
    for i in range(NCHUNK):
        out = _project(wes[i], tt3[i * Bc:(i + 1) * Bc], p2, dtg, wg,
                       out, i * (Bc // BB), B)
    return out


# speedup vs baseline: 1.0378x; 1.0378x over previous
import functools

import jax
import jax.numpy as jnp
from jax import lax
from jax.experimental import pallas as pl
from jax.experimental.pallas import tpu as pltpu
from jax.experimental.pallas import tpu_sc as plsc

NC = 2
NS = 16
NW = NC * NS
CHUNK = 128


def _gather_rows(table, ids, emb, dtype):
    n = ids.shape[0]
    per_w = n // NW
    ch = per_w // CHUNK
    ids3 = ids.reshape(NW, ch, CHUNK)
    mesh = plsc.VectorSubcoreMesh(core_axis_name="c", subcore_axis_name="s")

    @functools.partial(
        pl.kernel,
        mesh=mesh,
        out_type=jax.ShapeDtypeStruct((n, emb), dtype),
        scratch_types=[
            pltpu.VMEM((ch, CHUNK), jnp.int32),
            pltpu.VMEM((CHUNK, emb), dtype),
            pltpu.VMEM((CHUNK, emb), dtype),
            pltpu.SemaphoreType.DMA,
            pltpu.SemaphoreType.DMA,
        ],
    )
    def k(table_hbm, idx_hbm, out_hbm, idx_v, buf0, buf1, sem0, sem1):
        wid = lax.axis_index("s") * NC + lax.axis_index("c")
        pltpu.sync_copy(idx_hbm.at[wid], idx_v)
        base = wid * per_w
        pltpu.async_copy(table_hbm.at[idx_v.at[0]], buf0, sem0)

        def pair(i, carry):
            j = 2 * i
            pltpu.async_copy(table_hbm.at[idx_v.at[j + 1]], buf1, sem1)
            pltpu.make_async_copy(table_hbm.at[idx_v.at[j]], buf0, sem0).wait()
            pltpu.sync_copy(buf0, out_hbm.at[pl.ds(base + j * CHUNK, CHUNK)])

            @pl.when(j + 2 < ch)
            def _():
                pltpu.async_copy(table_hbm.at[idx_v.at[j + 2]], buf0, sem0)

            pltpu.make_async_copy(table_hbm.at[idx_v.at[j + 1]], buf1, sem1).wait()
            pltpu.sync_copy(buf1, out_hbm.at[pl.ds(base + (j + 1) * CHUNK, CHUNK)])
            return carry

        lax.fori_loop(0, ch // 2, pair, 0)

    return k(table, ids3)


def _project(we, tt3, p2, wg):
    B, S, emb = we.shape
    hid = wg.shape[1]

    BB = 16

    def body(we_ref, tt_ref, p2_ref, w_ref, out_ref):
        zero = jnp.zeros((1, emb), jnp.bfloat16)
        for q in range(BB):
            x = we_ref[q].astype(jnp.bfloat16)
            up = jnp.concatenate([x[1:], zero], axis=0)
            down = jnp.concatenate([zero, x[:-1]], axis=0)
            ttb = tt_ref[q].astype(jnp.bfloat16)
            cat = jnp.concatenate([up, x, down, ttb], axis=1)
            acc = jnp.dot(cat, w_ref[...], preferred_element_type=jnp.float32)
            out_ref[q] = acc + p2_ref[...]

    return pl.pallas_call(
        body,
        grid=(B // BB,),
        in_specs=[
            pl.BlockSpec((BB, S, emb), lambda i: (i, 0, 0)),
            pl.BlockSpec((BB, S, 1), lambda i: (i, 0, 0)),
            pl.BlockSpec((S, hid), lambda i: (0, 0)),
            pl.BlockSpec(wg.shape, lambda i: (0, 0)),
        ],
        out_specs=pl.BlockSpec((BB, S, hid), lambda i: (i, 0, 0)),
        out_shape=jax.ShapeDtypeStruct((B, S, hid), jnp.float32),
    )(we, tt3, p2, wg)


def kernel(input_ids, token_type_ids, word_table, pos_table, type_table, W, b,
           ln_weight, ln_bias):
    B, S = input_ids.shape
    emb = word_table.shape[1]
    hid = W.shape[1]
    ids = input_ids.astype(jnp.int32).reshape(B * S)
    tt3 = token_type_ids.astype(jnp.int32).reshape(B, S, 1)
    g = ln_weight.reshape(1, hid)
    p2 = (pos_table + b.reshape(1, hid) + type_table[0].reshape(1, hid)) * g \
        + ln_bias.reshape(1, hid)
    dtg = ((type_table[1] - type_table[0]).reshape(1, hid) * g)
    wg = jnp.concatenate([W * g, dtg], axis=0).astype(jnp.bfloat16)
    we = _gather_rows(word_table, ids, emb, jnp.float32).reshape(B, S, emb)
    out = _project(we, tt3, p2, wg)
    return out
